# Initial kernel scaffold; baseline (speedup 1.0000x reference)
import jax
import jax.numpy as jnp
from jax.experimental import pallas as pl


def kernel(x, beat_info, pitch_emb, beat_emb, dur_emb):
    raise NotImplementedError("write your pallas kernel here")


# speedup vs baseline: 2.4866x; 2.4866x over previous
import functools

import jax
import jax.numpy as jnp
from jax import lax
from jax.experimental import pallas as pl
from jax.experimental.pallas import tpu as pltpu
from jax.experimental.pallas import tpu_sc as plsc

_B = 1024
_L = 200
_EMB = 128
_CHUNK = 128
_CAT_ROWS = _B * 2 * _L
_BE_ROWS = _B * _L
_CAT_CHUNKS = _CAT_ROWS // _CHUNK
_BE_CHUNKS = _BE_ROWS // _CHUNK
_NC = 2
_NS = 16
_NW = _NC * _NS
_CAT_PER_W = _CAT_CHUNKS // _NW
_BE_PER_W = _BE_CHUNKS // _NW


@functools.lru_cache(maxsize=1)
def _make_lookup():
  mesh = plsc.VectorSubcoreMesh(
      core_axis_name="c", subcore_axis_name="s", num_cores=_NC)

  @functools.partial(
      pl.kernel,
      mesh=mesh,
      out_type=[
          jax.ShapeDtypeStruct((_CAT_ROWS, _EMB), jnp.float32),
          jax.ShapeDtypeStruct((_BE_ROWS, _EMB), jnp.float32),
      ],
      scratch_types=[
          pltpu.VMEM((_CHUNK,), jnp.int32),
          pltpu.VMEM((_CHUNK, _EMB), jnp.float32),
          pltpu.SemaphoreType.DMA,
      ],
  )
  def lookup(table_hbm, idxcat_hbm, idxbe_hbm, outcat_hbm, outbe_hbm,
             idx_v, rows_v, sem):
    wid = lax.axis_index("s") * _NC + lax.axis_index("c")

    def field(idx_hbm, out_hbm, per_w):
      base = wid * per_w

      def body(i, carry):
        c = base + i
        pltpu.sync_copy(idx_hbm.at[c], idx_v)
        pltpu.async_copy(table_hbm.at[idx_v], rows_v, sem).wait()
        pltpu.sync_copy(rows_v, out_hbm.at[pl.ds(c * _CHUNK, _CHUNK)])
        return carry

      lax.fori_loop(0, per_w, body, 0)

    field(idxcat_hbm, outcat_hbm, _CAT_PER_W)
    field(idxbe_hbm, outbe_hbm, _BE_PER_W)

  return lookup


def kernel(x, beat_info, pitch_emb, beat_emb, dur_emb):
  pitch = x[..., 2]
  dur = x[..., 3]
  off_dur = pitch_emb.shape[0]
  off_beat = off_dur + dur_emb.shape[0]
  table = jnp.concatenate([pitch_emb, dur_emb, beat_emb], axis=0)
  idx_cat = jnp.concatenate([pitch, dur + off_dur], axis=1).reshape(
      _CAT_CHUNKS, _CHUNK)
  idx_be = (beat_info + off_beat).reshape(_BE_CHUNKS, _CHUNK)
  out_cat_flat, be_flat = _make_lookup()(table, idx_cat, idx_be)
  out_cat = out_cat_flat.reshape(_B, 2 * _L, _EMB)
  be = be_flat.reshape(_B, _L, _EMB)
  return (out_cat, be, beat_info, pitch, dur)


# speedup vs baseline: 2.5197x; 1.0133x over previous
import functools

import jax
import jax.numpy as jnp
from jax import lax
from jax.experimental import pallas as pl
from jax.experimental.pallas import tpu as pltpu
from jax.experimental.pallas import tpu_sc as plsc

_B = 1024
_L = 200
_EMB = 128
_CHUNK = 128
_CAT_ROWS = _B * 2 * _L
_BE_ROWS = _B * _L
_CAT_CHUNKS = _CAT_ROWS // _CHUNK
_BE_CHUNKS = _BE_ROWS // _CHUNK
_NC = 2
_NS = 16
_NW = _NC * _NS
_CAT_PER_W = _CAT_CHUNKS // _NW
_BE_PER_W = _BE_CHUNKS // _NW
_IDX_ROWS = _CAT_PER_W + _BE_PER_W


@functools.lru_cache(maxsize=1)
def _make_lookup():
  mesh = plsc.VectorSubcoreMesh(
      core_axis_name="c", subcore_axis_name="s", num_cores=_NC)

  @functools.partial(
      pl.kernel,
      mesh=mesh,
      out_type=[
          jax.ShapeDtypeStruct((_CAT_ROWS, _EMB), jnp.float32),
          jax.ShapeDtypeStruct((_BE_ROWS, _EMB), jnp.float32),
      ],
      scratch_types=[
          pltpu.VMEM((_CAT_PER_W, _CHUNK), jnp.int32),
          pltpu.VMEM((_BE_PER_W, _CHUNK), jnp.int32),
          pltpu.VMEM((_CHUNK, _EMB), jnp.float32),
          pltpu.VMEM((_CHUNK, _EMB), jnp.float32),
          pltpu.SemaphoreType.DMA,
          pltpu.SemaphoreType.DMA,
      ],
  )
  def lookup(table_hbm, idxcat_hbm, idxbe_hbm, outcat_hbm, outbe_hbm,
             idxc_v, idxb_v, buf0, buf1, sem0, sem1):
    wid = lax.axis_index("s") * _NC + lax.axis_index("c")

    pltpu.sync_copy(idxcat_hbm.at[wid], idxc_v)
    pltpu.sync_copy(idxbe_hbm.at[wid], idxb_v)

    def fire(idx_v, j, buf, sem):
      pltpu.async_copy(table_hbm.at[idx_v.at[j]], buf, sem)

    def drain(idx_v, buf, sem):
      pltpu.make_async_copy(table_hbm.at[idx_v.at[0]], buf, sem).wait()

    def field(idx_v, out_hbm, per_w):
      base = wid * per_w
      fire(idx_v, 0, buf0, sem0)
      fire(idx_v, 1, buf1, sem1)

      def body(k, carry):
        i0 = 2 * k
        i1 = i0 + 1
        drain(idx_v, buf0, sem0)
        pltpu.sync_copy(buf0, out_hbm.at[pl.ds((base + i0) * _CHUNK, _CHUNK)])

        @pl.when(i0 + 2 < per_w)
        def _():
          fire(idx_v, i0 + 2, buf0, sem0)

        drain(idx_v, buf1, sem1)
        pltpu.sync_copy(buf1, out_hbm.at[pl.ds((base + i1) * _CHUNK, _CHUNK)])

        @pl.when(i1 + 2 < per_w)
        def _():
          fire(idx_v, i1 + 2, buf1, sem1)

        return carry

      lax.fori_loop(0, per_w // 2, body, 0)

    field(idxc_v, outcat_hbm, _CAT_PER_W)
    field(idxb_v, outbe_hbm, _BE_PER_W)

  return lookup


def kernel(x, beat_info, pitch_emb, beat_emb, dur_emb):
  pitch = x[..., 2]
  dur = x[..., 3]
  off_dur = pitch_emb.shape[0]
  off_beat = off_dur + dur_emb.shape[0]
  table = jnp.concatenate([pitch_emb, dur_emb, beat_emb], axis=0)
  idx_cat = jnp.concatenate([pitch, dur + off_dur], axis=1).reshape(
      _NW, _CAT_PER_W, _CHUNK)
  idx_be = (beat_info + off_beat).reshape(_NW, _BE_PER_W, _CHUNK)
  out_cat_flat, be_flat = _make_lookup()(table, idx_cat, idx_be)
  out_cat = out_cat_flat.reshape(_B, 2 * _L, _EMB)
  be = be_flat.reshape(_B, _L, _EMB)
  return (out_cat, be, beat_info, pitch, dur)


# speedup vs baseline: 15.1159x; 5.9991x over previous
import functools

import jax
import jax.numpy as jnp
from jax import lax
from jax.experimental import pallas as pl
from jax.experimental.pallas import tpu as pltpu
from jax.experimental.pallas import tpu_sc as plsc

_B = 1024
_L = 200
_EMB = 128
_CHUNK = 128
_CAT_ROWS = _B * 2 * _L
_BE_ROWS = _B * _L
_CAT_CHUNKS = _CAT_ROWS // _CHUNK
_BE_CHUNKS = _BE_ROWS // _CHUNK
_NC = 2
_NS = 16
_NW = _NC * _NS
_CAT_PER_W = _CAT_CHUNKS // _NW
_BE_PER_W = _BE_CHUNKS // _NW
_IDX_ROWS = _CAT_PER_W + _BE_PER_W
_TAB = 256


@functools.lru_cache(maxsize=1)
def _make_lookup():
  mesh = plsc.VectorSubcoreMesh(
      core_axis_name="c", subcore_axis_name="s", num_cores=_NC)

  @functools.partial(
      pl.kernel,
      mesh=mesh,
      out_type=[
          jax.ShapeDtypeStruct((_CAT_ROWS, _EMB), jnp.float32),
          jax.ShapeDtypeStruct((_BE_ROWS, _EMB), jnp.float32),
      ],
      scratch_types=[
          pltpu.VMEM((_CAT_PER_W, _CHUNK), jnp.int32),
          pltpu.VMEM((_BE_PER_W, _CHUNK), jnp.int32),
          pltpu.VMEM((_CHUNK, _EMB), jnp.float32),
          pltpu.VMEM((_CHUNK, _EMB), jnp.float32),
          pltpu.VMEM_SHARED((_TAB, _EMB), jnp.float32),
          pltpu.SemaphoreType.DMA,
          pltpu.SemaphoreType.DMA,
      ],
  )
  def lookup(table_hbm, idxcat_hbm, idxbe_hbm, outcat_hbm, outbe_hbm,
             idxc_v, idxb_v, buf0, buf1, table_sp, sem0, sem1):
    sid = lax.axis_index("s")
    wid = sid * _NC + lax.axis_index("c")

    @pl.when(sid == 0)
    def _():
      pltpu.sync_copy(table_hbm, table_sp)

    plsc.subcore_barrier()

    pltpu.sync_copy(idxcat_hbm.at[wid], idxc_v)
    pltpu.sync_copy(idxbe_hbm.at[wid], idxb_v)

    def fire(idx_v, j, buf, sem):
      pltpu.async_copy(table_sp.at[idx_v.at[j]], buf, sem)

    def drain(idx_v, buf, sem):
      pltpu.make_async_copy(table_sp.at[idx_v.at[0]], buf, sem).wait()

    def field(idx_v, out_hbm, per_w):
      base = wid * per_w
      fire(idx_v, 0, buf0, sem0)
      fire(idx_v, 1, buf1, sem1)

      def body(k, carry):
        i0 = 2 * k
        i1 = i0 + 1
        drain(idx_v, buf0, sem0)
        pltpu.sync_copy(buf0, out_hbm.at[pl.ds((base + i0) * _CHUNK, _CHUNK)])

        @pl.when(i0 + 2 < per_w)
        def _():
          fire(idx_v, i0 + 2, buf0, sem0)

        drain(idx_v, buf1, sem1)
        pltpu.sync_copy(buf1, out_hbm.at[pl.ds((base + i1) * _CHUNK, _CHUNK)])

        @pl.when(i1 + 2 < per_w)
        def _():
          fire(idx_v, i1 + 2, buf1, sem1)

        return carry

      lax.fori_loop(0, per_w // 2, body, 0)

    field(idxc_v, outcat_hbm, _CAT_PER_W)
    field(idxb_v, outbe_hbm, _BE_PER_W)

  return lookup


def kernel(x, beat_info, pitch_emb, beat_emb, dur_emb):
  pitch = x[..., 2]
  dur = x[..., 3]
  off_dur = pitch_emb.shape[0]
  off_beat = off_dur + dur_emb.shape[0]
  table = jnp.concatenate([pitch_emb, dur_emb, beat_emb], axis=0)
  idx_cat = jnp.concatenate([pitch, dur + off_dur], axis=1).reshape(
      _NW, _CAT_PER_W, _CHUNK)
  idx_be = (beat_info + off_beat).reshape(_NW, _BE_PER_W, _CHUNK)
  out_cat_flat, be_flat = _make_lookup()(table, idx_cat, idx_be)
  out_cat = out_cat_flat.reshape(_B, 2 * _L, _EMB)
  be = be_flat.reshape(_B, _L, _EMB)
  return (out_cat, be, beat_info, pitch, dur)
